# baseline (speedup 1.0000x reference)
import functools

import jax
import jax.numpy as jnp
from jax.experimental import pallas as pl
from jax.experimental.pallas import tpu as pltpu


def _router_body(bn, thr, xt_ref, wt_ref, wm_ref, out_ref, wbuf, anyf):
    p = pl.program_id(0)
    i = pl.program_id(1)
    e = out_ref.shape[0]
    k = wt_ref.shape[0] // e

    @pl.when(p == 0)
    def _compute_weights():
        xtb = xt_ref[...]
        yt = jnp.dot(wt_ref[...], xtb, preferred_element_type=jnp.float32)
        rows = []
        for ei in range(e):
            projt = jnp.dot(wm_ref[:, ei * k:(ei + 1) * k],
                            yt[ei * k:(ei + 1) * k, :],
                            preferred_element_type=jnp.float32)
            d2 = jnp.sum((xtb - projt) ** 2, axis=0, keepdims=True)
            rows.append(jnp.sqrt(d2))
        resid = jnp.concatenate(rows, axis=0)
        logits = -resid
        m = jnp.max(logits, axis=0, keepdims=True)
        ex = jnp.exp(logits - m)
        w = ex / jnp.sum(ex, axis=0, keepdims=True)
        wbuf[:, pl.ds(i * bn, bn)] = w
        blk_any = jnp.any(w > thr).astype(jnp.int32)
        anyf[0] = jnp.where(i == 0, blk_any, jnp.maximum(anyf[0], blk_any))

    @pl.when(p == 1)
    def _finalize():
        w = wbuf[:, pl.ds(i * bn, bn)]
        any_v = anyf[0] != 0
        idx = jax.lax.broadcasted_iota(jnp.int32, w.shape, 0)
        m1 = jnp.max(w, axis=0, keepdims=True)
        i1 = jnp.min(jnp.where(w == m1, idx, e), axis=0, keepdims=True)
        oh1 = idx == i1
        w2 = jnp.where(oh1, -jnp.inf, w)
        m2 = jnp.max(w2, axis=0, keepdims=True)
        i2 = jnp.min(jnp.where(w2 == m2, idx, e), axis=0, keepdims=True)
        tk = oh1 | (idx == i2)
        mask = (((w > thr) & any_v) | (oh1 & jnp.logical_not(any_v))) & tk
        filt = jnp.where(mask, w, 0.0)
        ssum = jnp.sum(filt, axis=0, keepdims=True)
        ssum = jnp.where(ssum == 0.0, 1.0, ssum)
        out_ref[...] = filt / ssum


def kernel(x, x_l, V):
    del x_l
    n, d = x.shape
    e, _, k = V.shape
    ek = e * k
    bn = 1024 if n % 1024 == 0 else n
    nblk = n // bn
    thr = 1.0 / e

    xt = jnp.transpose(x)
    wmat = jnp.transpose(V, (1, 0, 2)).reshape(d, ek)
    wt = jnp.transpose(wmat)

    body = functools.partial(_router_body, bn, thr)
    out_t = pl.pallas_call(
        body,
        grid=(2, nblk),
        in_specs=[
            pl.BlockSpec((d, bn), lambda p, i: (0, i * (1 - p))),
            pl.BlockSpec((ek, d), lambda p, i: (0, 0)),
            pl.BlockSpec((d, ek), lambda p, i: (0, 0)),
        ],
        out_specs=pl.BlockSpec((e, bn), lambda p, i: (0, i)),
        out_shape=jax.ShapeDtypeStruct((e, n), jnp.float32),
        scratch_shapes=[
            pltpu.VMEM((e, n), jnp.float32),
            pltpu.SMEM((1,), jnp.int32),
        ],
        compiler_params=pltpu.CompilerParams(
            vmem_limit_bytes=100 * 1024 * 1024,
        ),
    )(xt, wt, wmat)
    return jnp.transpose(out_t)


# speedup vs baseline: 1.4743x; 1.4743x over previous
import functools

import jax
import jax.numpy as jnp
from jax.experimental import pallas as pl
from jax.experimental.pallas import tpu as pltpu


def _router_body(bn, thr, xt_ref, wt_ref, wm_ref, out_ref, wbuf, anyf):
    p = pl.program_id(0)
    i = pl.program_id(1)
    e = out_ref.shape[0]
    k = wt_ref.shape[0] // e

    @pl.when(p == 0)
    def _compute_weights():
        xtb = jnp.transpose(xt_ref[...])
        yt = jnp.dot(wt_ref[...], xtb, preferred_element_type=jnp.float32)
        rows = []
        for ei in range(e):
            projt = jnp.dot(wm_ref[:, ei * k:(ei + 1) * k],
                            yt[ei * k:(ei + 1) * k, :],
                            preferred_element_type=jnp.float32)
            d2 = jnp.sum((xtb - projt) ** 2, axis=0, keepdims=True)
            rows.append(jnp.sqrt(d2))
        resid = jnp.concatenate(rows, axis=0)
        logits = -resid
        m = jnp.max(logits, axis=0, keepdims=True)
        ex = jnp.exp(logits - m)
        w = ex / jnp.sum(ex, axis=0, keepdims=True)
        wbuf[:, pl.ds(i * bn, bn)] = w
        blk_any = jnp.any(w > thr).astype(jnp.int32)
        anyf[0] = jnp.where(i == 0, blk_any, jnp.maximum(anyf[0], blk_any))

    @pl.when(p == 1)
    def _finalize():
        w = wbuf[:, pl.ds(i * bn, bn)]
        any_v = anyf[0] != 0
        idx = jax.lax.broadcasted_iota(jnp.int32, w.shape, 0)
        m1 = jnp.max(w, axis=0, keepdims=True)
        i1 = jnp.min(jnp.where(w == m1, idx, e), axis=0, keepdims=True)
        oh1 = idx == i1
        w2 = jnp.where(oh1, -jnp.inf, w)
        m2 = jnp.max(w2, axis=0, keepdims=True)
        i2 = jnp.min(jnp.where(w2 == m2, idx, e), axis=0, keepdims=True)
        tk = oh1 | (idx == i2)
        mask = (((w > thr) & any_v) | (oh1 & jnp.logical_not(any_v))) & tk
        filt = jnp.where(mask, w, 0.0)
        ssum = jnp.sum(filt, axis=0, keepdims=True)
        ssum = jnp.where(ssum == 0.0, 1.0, ssum)
        out_ref[...] = filt / ssum


def kernel(x, x_l, V):
    del x_l
    n, d = x.shape
    e, _, k = V.shape
    ek = e * k
    bn = 1024 if n % 1024 == 0 else n
    nblk = n // bn
    thr = 1.0 / e

    wmat = jnp.transpose(V, (1, 0, 2)).reshape(d, ek)
    wt = jnp.transpose(wmat)

    body = functools.partial(_router_body, bn, thr)
    out_t = pl.pallas_call(
        body,
        grid=(2, nblk),
        in_specs=[
            pl.BlockSpec((bn, d), lambda p, i: (i * (1 - p), 0)),
            pl.BlockSpec((ek, d), lambda p, i: (0, 0)),
            pl.BlockSpec((d, ek), lambda p, i: (0, 0)),
        ],
        out_specs=pl.BlockSpec((e, bn), lambda p, i: (0, i)),
        out_shape=jax.ShapeDtypeStruct((e, n), jnp.float32),
        scratch_shapes=[
            pltpu.VMEM((e, n), jnp.float32),
            pltpu.SMEM((1,), jnp.int32),
        ],
        compiler_params=pltpu.CompilerParams(
            vmem_limit_bytes=100 * 1024 * 1024,
        ),
    )(x, wt, wmat)
    return jnp.transpose(out_t)
